# Initial kernel scaffold; baseline (speedup 1.0000x reference)
import jax
import jax.numpy as jnp
from jax.experimental import pallas as pl


def kernel(x, edge_index, edge_attr, W_enc, as_enc, ad_enc, b_enc, W_h0, as_h0, ad_h0, b_h0, W_h1, as_h1, ad_h1, b_h1, W_h2, as_h2, ad_h2, b_h2, W_dec, as_dec, ad_dec, b_dec):
    raise NotImplementedError("write your pallas kernel here")


# speedup vs baseline: 27.7891x; 27.7891x over previous
import functools

import jax
import jax.numpy as jnp
from jax import lax
from jax.experimental import pallas as pl
from jax.experimental.pallas import tpu as pltpu
from jax.experimental.pallas import tpu_sc as plsc

N = 10000
E = 320000
F = 128
E_REAL = E + N
NC = 2
NS = 16
NW = NC * NS
K = 128
PER_W = 10368
E_PAD = PER_W * NW
CHUNKS = PER_W // K
N_PAD = 10112
ROWS_PER_TILE = N_PAD // NS
BN = 2000
EPS = 1e-16


def _tc_first_body(x_ref, w_ref, a_ref, h_ref, al_ref, cm_ref):
    i = pl.program_id(0)
    h = jnp.dot(x_ref[...], w_ref[...], preferred_element_type=jnp.float32)
    h_ref[...] = h
    al = jnp.dot(h, a_ref[...], preferred_element_type=jnp.float32)
    al_ref[...] = al
    cm = jnp.max(al[:, :16], axis=0, keepdims=True)

    @pl.when(i == 0)
    def _():
        cm_ref[...] = cm

    @pl.when(i != 0)
    def _():
        cm_ref[...] = jnp.maximum(cm_ref[...], cm)


def _tc_first(x, W, Acat):
    nblk = N // BN
    return pl.pallas_call(
        _tc_first_body,
        grid=(nblk,),
        in_specs=[
            pl.BlockSpec((BN, F), lambda i: (i, 0)),
            pl.BlockSpec((F, F), lambda i: (0, 0)),
            pl.BlockSpec((F, F), lambda i: (0, 0)),
        ],
        out_specs=[
            pl.BlockSpec((BN, F), lambda i: (i, 0)),
            pl.BlockSpec((BN, F), lambda i: (i, 0)),
            pl.BlockSpec((1, 16), lambda i: (0, 0)),
        ],
        out_shape=[
            jax.ShapeDtypeStruct((N, F), jnp.float32),
            jax.ShapeDtypeStruct((N, F), jnp.float32),
            jax.ShapeDtypeStruct((1, 16), jnp.float32),
        ],
    )(x, W, Acat)


def _merge_prev(o0_ref, o1_ref, s_ref, b_ref, bm_ref):
    factor = 1.0 / (s_ref[...] + EPS)
    expand = jnp.dot(factor, bm_ref[...], preferred_element_type=jnp.float32)
    return (o0_ref[...] + o1_ref[...]) * expand + b_ref[...]


def _tc_mid_body(act, o0_ref, o1_ref, s_ref, b_ref, bm_ref,
                 w_ref, a_ref, h_ref, al_ref, cm_ref):
    i = pl.program_id(0)
    o = _merge_prev(o0_ref, o1_ref, s_ref, b_ref, bm_ref)
    if act:
        o = jnp.maximum(o, 0.01 * o)
    h = jnp.dot(o, w_ref[...], preferred_element_type=jnp.float32)
    h_ref[...] = h
    al = jnp.dot(h, a_ref[...], preferred_element_type=jnp.float32)
    al_ref[...] = al
    cm = jnp.max(al[:, :16], axis=0, keepdims=True)

    @pl.when(i == 0)
    def _():
        cm_ref[...] = cm

    @pl.when(i != 0)
    def _():
        cm_ref[...] = jnp.maximum(cm_ref[...], cm)


def _tc_mid(o0, o1, s, b, Bmask, W, Acat, act):
    nblk = N // BN
    return pl.pallas_call(
        functools.partial(_tc_mid_body, act),
        grid=(nblk,),
        in_specs=[
            pl.BlockSpec((BN, F), lambda i: (i, 0)),
            pl.BlockSpec((BN, F), lambda i: (i, 0)),
            pl.BlockSpec((BN, 8), lambda i: (i, 0)),
            pl.BlockSpec((1, F), lambda i: (0, 0)),
            pl.BlockSpec((8, F), lambda i: (0, 0)),
            pl.BlockSpec((F, F), lambda i: (0, 0)),
            pl.BlockSpec((F, F), lambda i: (0, 0)),
        ],
        out_specs=[
            pl.BlockSpec((BN, F), lambda i: (i, 0)),
            pl.BlockSpec((BN, F), lambda i: (i, 0)),
            pl.BlockSpec((1, 16), lambda i: (0, 0)),
        ],
        out_shape=[
            jax.ShapeDtypeStruct((N, F), jnp.float32),
            jax.ShapeDtypeStruct((N, F), jnp.float32),
            jax.ShapeDtypeStruct((1, 16), jnp.float32),
        ],
    )(o0, o1, s, b, Bmask, W, Acat)


def _tc_final_body(o0_ref, o1_ref, s_ref, b_ref, bm_ref, out_ref):
    out_ref[...] = _merge_prev(o0_ref, o1_ref, s_ref, b_ref, bm_ref)


def _tc_final(o0, o1, s, b, Bmask):
    nblk = N // BN
    return pl.pallas_call(
        _tc_final_body,
        grid=(nblk,),
        in_specs=[
            pl.BlockSpec((BN, F), lambda i: (i, 0)),
            pl.BlockSpec((BN, F), lambda i: (i, 0)),
            pl.BlockSpec((BN, 8), lambda i: (i, 0)),
            pl.BlockSpec((1, F), lambda i: (0, 0)),
            pl.BlockSpec((8, F), lambda i: (0, 0)),
        ],
        out_specs=pl.BlockSpec((BN, F), lambda i: (i, 0)),
        out_shape=jax.ShapeDtypeStruct((N, F), jnp.float32),
    )(o0, o1, s, b, Bmask)


_SC_PARAMS = pltpu.CompilerParams(needs_layout_passes=False)


def _sc_a_body(H, al_hbm, src_hbm, dst_hbm, c_hbm, zs_hbm,
               ee_out, s_out,
               s_acc, srcv, dstv, als, ald, eev, cvec, gsem, gsem2):
    core = lax.axis_index("c")
    sub = lax.axis_index("s")
    wid = sub * NC + core

    pltpu.sync_copy(zs_hbm, s_acc)
    pltpu.sync_copy(c_hbm, cvec)

    iota = lax.iota(jnp.int32, 16)
    lane_hi = lax.shift_right_logical(iota, 3)
    lane_h = lax.bitwise_and(iota, 7)
    cv = cvec[...]
    base = wid * PER_W

    def chunk(i, _):
        off = base + i * K
        pltpu.sync_copy(src_hbm.at[pl.ds(off, K)], srcv)
        pltpu.sync_copy(dst_hbm.at[pl.ds(off, K)], dstv)
        pltpu.async_copy(al_hbm.at[srcv], als, gsem).wait()
        pltpu.async_copy(al_hbm.at[dstv], ald, gsem2).wait()

        def eslice(t, _):
            row = lane_hi + 2 * t
            vs = plsc.load_gather(als, [row, lane_h])
            vd = plsc.load_gather(ald, [row, lane_h + H])
            v = vs + vd
            ev = jnp.maximum(v, 0.2 * v)
            ee = jnp.exp(ev - cv)
            valid = (off + 2 * t) + lane_hi < E_REAL
            if H < 8:
                valid = valid & (lane_h < H)
            ee = jnp.where(valid, ee, 0.0)
            eev[pl.ds(16 * t, 16)] = ee
            nodes = plsc.load_gather(dstv, [row])
            plsc.addupdate_scatter(
                s_acc, [lax.shift_left(nodes, 3) + lane_h], ee)
            return 0

        lax.fori_loop(0, K // 2, eslice, 0, unroll=4)
        pltpu.sync_copy(eev, ee_out.at[pl.ds(off * 8, K * 8)])
        return 0

    lax.fori_loop(0, CHUNKS, chunk, 0)
    pltpu.sync_copy(s_acc, s_out.at[pl.ds(wid * (N_PAD * 8), N_PAD * 8)])


def _sc_a(H, alcat, src, dst, c16, zs):
    mesh = plsc.VectorSubcoreMesh(core_axis_name="c", subcore_axis_name="s")
    kfn = pl.kernel(
        functools.partial(_sc_a_body, H),
        out_type=(
            jax.ShapeDtypeStruct((E_PAD * 8,), jnp.float32),
            jax.ShapeDtypeStruct((NW * N_PAD * 8,), jnp.float32),
        ),
        mesh=mesh,
        compiler_params=_SC_PARAMS,
        scratch_types=[
            pltpu.VMEM((N_PAD * 8,), jnp.float32),
            pltpu.VMEM((K,), jnp.int32),
            pltpu.VMEM((K,), jnp.int32),
            pltpu.VMEM((K, F), jnp.float32),
            pltpu.VMEM((K, F), jnp.float32),
            pltpu.VMEM((K * 8,), jnp.float32),
            pltpu.VMEM((16,), jnp.float32),
            pltpu.SemaphoreType.DMA,
            pltpu.SemaphoreType.DMA,
        ],
    )
    return kfn(alcat, src, dst, c16, zs)


def _sc_b_body(H, h_hbm, ee_hbm, src_hbm, dst_hbm, zf_hbm, sp_hbm,
               acc_out, ssum_out,
               acc_sh, srcv, dstv, eev, hrows, ssum, stmp, gsem):
    D = F // H
    core = lax.axis_index("c")
    sub = lax.axis_index("s")
    wid = sub * NC + core
    row0 = sub * ROWS_PER_TILE
    sl = pl.ds(row0, ROWS_PER_TILE)
    RT8 = ROWS_PER_TILE * 8

    pltpu.sync_copy(zf_hbm.at[sl], acc_sh.at[sl])

    @pl.when(core == 0)
    def _():
        pltpu.sync_copy(sp_hbm.at[pl.ds(sub * RT8, RT8)], ssum)

        def accw(w, _):
            pltpu.sync_copy(
                sp_hbm.at[pl.ds(w * (N_PAD * 8) + sub * RT8, RT8)], stmp)

            def vadd(j, _):
                jsl = pl.ds(j * 16, 16)
                ssum[jsl] = ssum[jsl] + stmp[jsl]
                return 0

            lax.fori_loop(0, RT8 // 16, vadd, 0, unroll=8)
            return 0

        lax.fori_loop(1, NW, accw, 0)
        pltpu.sync_copy(ssum, ssum_out.at[pl.ds(sub * RT8, RT8)])

    plsc.subcore_barrier()

    iota = lax.iota(jnp.int32, 16)
    base = wid * PER_W

    def chunk(i, _):
        off = base + i * K
        pltpu.sync_copy(src_hbm.at[pl.ds(off, K)], srcv)
        pltpu.sync_copy(dst_hbm.at[pl.ds(off, K)], dstv)
        pltpu.sync_copy(ee_hbm.at[pl.ds(off * 8, K * 8)], eev)
        pltpu.async_copy(h_hbm.at[srcv], hrows, gsem).wait()

        def wrow(k, _):
            rowk = jnp.full((16,), k, jnp.int32)
            rowk8 = lax.shift_left(rowk, 3)
            for m in range(8):
                hd = (16 * m) // D
                colh = iota + (16 * m)
                a = plsc.load_gather(eev, [rowk8 + hd])
                hv = plsc.load_gather(hrows, [rowk, colh])
                plsc.store_scatter(hrows, [rowk, colh], hv * a)
            return 0

        lax.fori_loop(0, K, wrow, 0, unroll=2)
        pltpu.sync_copy(hrows, acc_sh.at[dstv], add=True)
        return 0

    lax.fori_loop(0, CHUNKS, chunk, 0)
    plsc.subcore_barrier()
    pltpu.sync_copy(acc_sh.at[sl], acc_out.at[core, sl])


def _sc_b(H, h, ee, src, dst, zf, s_parts):
    mesh = plsc.VectorSubcoreMesh(core_axis_name="c", subcore_axis_name="s")
    kfn = pl.kernel(
        functools.partial(_sc_b_body, H),
        out_type=(
            jax.ShapeDtypeStruct((NC, N_PAD, F), jnp.float32),
            jax.ShapeDtypeStruct((N_PAD * 8,), jnp.float32),
        ),
        mesh=mesh,
        compiler_params=_SC_PARAMS,
        scratch_types=[
            pltpu.VMEM_SHARED((N_PAD, F), jnp.float32),
            pltpu.VMEM((K,), jnp.int32),
            pltpu.VMEM((K,), jnp.int32),
            pltpu.VMEM((K * 8,), jnp.float32),
            pltpu.VMEM((K, F), jnp.float32),
            pltpu.VMEM((ROWS_PER_TILE * 8,), jnp.float32),
            pltpu.VMEM((ROWS_PER_TILE * 8,), jnp.float32),
            pltpu.SemaphoreType.DMA,
        ],
    )
    return kfn(h, ee, src, dst, zf, s_parts)


def _make_acat(a_s, a_d):
    Hh, Dd = a_s.shape
    rows = jnp.arange(F)
    onehot = (rows[:, None] // Dd == jnp.arange(Hh)[None, :]).astype(jnp.float32)
    As = onehot * a_s.reshape(F, 1)
    Ad = onehot * a_d.reshape(F, 1)
    cat = jnp.concatenate([As, Ad], axis=1)
    return jnp.pad(cat, ((0, 0), (0, F - cat.shape[1])))


def _make_c16(colmax, H):
    ms = colmax[0, :H]
    md = colmax[0, H:2 * H]
    c = ms + md
    c = jnp.maximum(c, 0.2 * c)
    return jnp.tile(c, 16 // H)


def _layer(H, h, alcat, cm, src, dst, zs, zf):
    c16 = _make_c16(cm, H)
    ee, s_parts = _sc_a(H, alcat, src, dst, c16, zs)
    o_p, s_sum = _sc_b(H, h, ee, src, dst, zf, s_parts)
    return s_sum.reshape(N_PAD, 8)[:N], o_p[:, :N]


def kernel(x, edge_index, edge_attr, W_enc, as_enc, ad_enc, b_enc,
           W_h0, as_h0, ad_h0, b_h0, W_h1, as_h1, ad_h1, b_h1,
           W_h2, as_h2, ad_h2, b_h2, W_dec, as_dec, ad_dec, b_dec):
    loop = jnp.arange(N, dtype=jnp.int32)
    pad = jnp.arange(E_PAD - E_REAL, dtype=jnp.int32)
    src = jnp.concatenate([edge_index[0].astype(jnp.int32), loop, pad])
    dst = jnp.concatenate([edge_index[1].astype(jnp.int32), loop, pad])

    zs = jnp.zeros((N_PAD * 8,), jnp.float32)
    zf = jnp.zeros((N_PAD, F), jnp.float32)
    bmask8 = jnp.repeat(jnp.eye(8, dtype=jnp.float32), 16, axis=1)
    bmask1 = jnp.concatenate(
        [jnp.ones((1, F), jnp.float32), jnp.zeros((7, F), jnp.float32)])

    h, alcat, cm = _tc_first(x, W_enc, _make_acat(as_enc, ad_enc))
    s_p, o_p = _layer(8, h, alcat, cm, src, dst, zs, zf)

    for (Wl, asl, adl), bprev, act in (
            ((W_h0, as_h0, ad_h0), b_enc, True),
            ((W_h1, as_h1, ad_h1), b_h0, False),
            ((W_h2, as_h2, ad_h2), b_h1, False),
    ):
        h, alcat, cm = _tc_mid(o_p[0], o_p[1], s_p, bprev.reshape(1, F),
                               bmask8, Wl, _make_acat(asl, adl), act)
        s_p, o_p = _layer(8, h, alcat, cm, src, dst, zs, zf)

    h, alcat, cm = _tc_mid(o_p[0], o_p[1], s_p, b_h2.reshape(1, F),
                           bmask8, W_dec, _make_acat(as_dec, ad_dec), False)
    s_p, o_p = _layer(1, h, alcat, cm, src, dst, zs, zf)

    return _tc_final(o_p[0], o_p[1], s_p, b_dec.reshape(1, F), bmask1)


# speedup vs baseline: 32.9572x; 1.1860x over previous
import functools

import jax
import jax.numpy as jnp
from jax import lax
from jax.experimental import pallas as pl
from jax.experimental.pallas import tpu as pltpu
from jax.experimental.pallas import tpu_sc as plsc

N = 10000
E = 320000
F = 128
E_REAL = E + N
NC = 2
NS = 16
NW = NC * NS
K = 128
PER_W = 10368
E_PAD = PER_W * NW
CHUNKS = PER_W // K
N_PAD = 10112
ROWS_PER_TILE = N_PAD // NS
BN = 2000
EPS = 1e-16


def _tc_first_body(x_ref, w_ref, a_ref, h_ref, al_ref, cm_ref):
    i = pl.program_id(0)
    h = jnp.dot(x_ref[...], w_ref[...], preferred_element_type=jnp.float32)
    h_ref[...] = h
    al = jnp.dot(h, a_ref[...], preferred_element_type=jnp.float32)
    al_ref[...] = al
    cm = jnp.max(al[:, :16], axis=0, keepdims=True)

    @pl.when(i == 0)
    def _():
        cm_ref[...] = cm

    @pl.when(i != 0)
    def _():
        cm_ref[...] = jnp.maximum(cm_ref[...], cm)


def _tc_first(x, W, Acat):
    nblk = N // BN
    return pl.pallas_call(
        _tc_first_body,
        grid=(nblk,),
        in_specs=[
            pl.BlockSpec((BN, F), lambda i: (i, 0)),
            pl.BlockSpec((F, F), lambda i: (0, 0)),
            pl.BlockSpec((F, F), lambda i: (0, 0)),
        ],
        out_specs=[
            pl.BlockSpec((BN, F), lambda i: (i, 0)),
            pl.BlockSpec((BN, F), lambda i: (i, 0)),
            pl.BlockSpec((1, 16), lambda i: (0, 0)),
        ],
        out_shape=[
            jax.ShapeDtypeStruct((N, F), jnp.float32),
            jax.ShapeDtypeStruct((N, F), jnp.float32),
            jax.ShapeDtypeStruct((1, 16), jnp.float32),
        ],
    )(x, W, Acat)


def _merge_prev(o0_ref, o1_ref, s_ref, b_ref, bm_ref):
    factor = 1.0 / (s_ref[...] + EPS)
    expand = jnp.dot(factor, bm_ref[...], preferred_element_type=jnp.float32)
    return (o0_ref[...] + o1_ref[...]) * expand + b_ref[...]


def _tc_mid_body(act, o0_ref, o1_ref, s_ref, b_ref, bm_ref,
                 w_ref, a_ref, h_ref, al_ref, cm_ref):
    i = pl.program_id(0)
    o = _merge_prev(o0_ref, o1_ref, s_ref, b_ref, bm_ref)
    if act:
        o = jnp.maximum(o, 0.01 * o)
    h = jnp.dot(o, w_ref[...], preferred_element_type=jnp.float32)
    h_ref[...] = h
    al = jnp.dot(h, a_ref[...], preferred_element_type=jnp.float32)
    al_ref[...] = al
    cm = jnp.max(al[:, :16], axis=0, keepdims=True)

    @pl.when(i == 0)
    def _():
        cm_ref[...] = cm

    @pl.when(i != 0)
    def _():
        cm_ref[...] = jnp.maximum(cm_ref[...], cm)


def _tc_mid(o0, o1, s, b, Bmask, W, Acat, act):
    nblk = N // BN
    return pl.pallas_call(
        functools.partial(_tc_mid_body, act),
        grid=(nblk,),
        in_specs=[
            pl.BlockSpec((BN, F), lambda i: (i, 0)),
            pl.BlockSpec((BN, F), lambda i: (i, 0)),
            pl.BlockSpec((BN, 8), lambda i: (i, 0)),
            pl.BlockSpec((1, F), lambda i: (0, 0)),
            pl.BlockSpec((8, F), lambda i: (0, 0)),
            pl.BlockSpec((F, F), lambda i: (0, 0)),
            pl.BlockSpec((F, F), lambda i: (0, 0)),
        ],
        out_specs=[
            pl.BlockSpec((BN, F), lambda i: (i, 0)),
            pl.BlockSpec((BN, F), lambda i: (i, 0)),
            pl.BlockSpec((1, 16), lambda i: (0, 0)),
        ],
        out_shape=[
            jax.ShapeDtypeStruct((N, F), jnp.float32),
            jax.ShapeDtypeStruct((N, F), jnp.float32),
            jax.ShapeDtypeStruct((1, 16), jnp.float32),
        ],
    )(o0, o1, s, b, Bmask, W, Acat)


def _tc_final_body(o0_ref, o1_ref, s_ref, b_ref, bm_ref, out_ref):
    out_ref[...] = _merge_prev(o0_ref, o1_ref, s_ref, b_ref, bm_ref)


def _tc_final(o0, o1, s, b, Bmask):
    nblk = N // BN
    return pl.pallas_call(
        _tc_final_body,
        grid=(nblk,),
        in_specs=[
            pl.BlockSpec((BN, F), lambda i: (i, 0)),
            pl.BlockSpec((BN, F), lambda i: (i, 0)),
            pl.BlockSpec((BN, 8), lambda i: (i, 0)),
            pl.BlockSpec((1, F), lambda i: (0, 0)),
            pl.BlockSpec((8, F), lambda i: (0, 0)),
        ],
        out_specs=pl.BlockSpec((BN, F), lambda i: (i, 0)),
        out_shape=jax.ShapeDtypeStruct((N, F), jnp.float32),
    )(o0, o1, s, b, Bmask)


_SC_PARAMS = pltpu.CompilerParams(needs_layout_passes=False)


def _sc_a_body(H, al_hbm, src_hbm, dst_hbm, c_hbm, zs_hbm,
               ee_out, s_out,
               s_acc, srcv, dstv, als, ald, eev, cvec, gsem, gsem2):
    core = lax.axis_index("c")
    sub = lax.axis_index("s")
    wid = sub * NC + core

    pltpu.sync_copy(zs_hbm, s_acc)
    pltpu.sync_copy(c_hbm, cvec)

    iota = lax.iota(jnp.int32, 16)
    lane_hi = lax.shift_right_logical(iota, 3)
    lane_h = lax.bitwise_and(iota, 7)
    cv = cvec[...]
    base = wid * PER_W

    def chunk(i, _):
        off = base + i * K
        d1 = pltpu.async_copy(src_hbm.at[pl.ds(off, K)], srcv, gsem)
        d2 = pltpu.async_copy(dst_hbm.at[pl.ds(off, K)], dstv, gsem2)
        d1.wait()
        d2.wait()
        g1 = pltpu.async_copy(al_hbm.at[srcv], als, gsem)
        g2 = pltpu.async_copy(al_hbm.at[dstv], ald, gsem2)
        g1.wait()
        g2.wait()

        def eslice(t, _):
            row = lane_hi + 2 * t
            vs = plsc.load_gather(als, [row, lane_h])
            vd = plsc.load_gather(ald, [row, lane_h + H])
            v = vs + vd
            ev = jnp.maximum(v, 0.2 * v)
            ee = jnp.exp(ev - cv)
            valid = (off + 2 * t) + lane_hi < E_REAL
            if H < 8:
                valid = valid & (lane_h < H)
            ee = jnp.where(valid, ee, 0.0)
            eev[pl.ds(16 * t, 16)] = ee
            nodes = plsc.load_gather(dstv, [row])
            plsc.addupdate_scatter(
                s_acc, [lax.shift_left(nodes, 3) + lane_h], ee)
            return 0

        lax.fori_loop(0, K // 2, eslice, 0, unroll=4)
        pltpu.sync_copy(eev, ee_out.at[pl.ds(off * 8, K * 8)])
        return 0

    lax.fori_loop(0, CHUNKS, chunk, 0)
    pltpu.sync_copy(s_acc, s_out.at[pl.ds(wid * (N_PAD * 8), N_PAD * 8)])


def _sc_a(H, alcat, src, dst, c16, zs):
    mesh = plsc.VectorSubcoreMesh(core_axis_name="c", subcore_axis_name="s")
    kfn = pl.kernel(
        functools.partial(_sc_a_body, H),
        out_type=(
            jax.ShapeDtypeStruct((E_PAD * 8,), jnp.float32),
            jax.ShapeDtypeStruct((NW * N_PAD * 8,), jnp.float32),
        ),
        mesh=mesh,
        compiler_params=_SC_PARAMS,
        scratch_types=[
            pltpu.VMEM((N_PAD * 8,), jnp.float32),
            pltpu.VMEM((K,), jnp.int32),
            pltpu.VMEM((K,), jnp.int32),
            pltpu.VMEM((K, F), jnp.float32),
            pltpu.VMEM((K, F), jnp.float32),
            pltpu.VMEM((K * 8,), jnp.float32),
            pltpu.VMEM((16,), jnp.float32),
            pltpu.SemaphoreType.DMA,
            pltpu.SemaphoreType.DMA,
        ],
    )
    return kfn(alcat, src, dst, c16, zs)


def _sc_b_body(H, h_hbm, ee_hbm, src_hbm, dst_hbm, zf_hbm, sp_hbm,
               acc_out, ssum_out,
               acc_sh, srcv, dstv, eev, hrows, ssum, stmp, gsem, gsem2,
               gsem3):
    D = F // H
    core = lax.axis_index("c")
    sub = lax.axis_index("s")
    wid = sub * NC + core
    row0 = sub * ROWS_PER_TILE
    sl = pl.ds(row0, ROWS_PER_TILE)
    RT8 = ROWS_PER_TILE * 8

    pltpu.sync_copy(zf_hbm.at[sl], acc_sh.at[sl])

    HT = RT8 // 2
    hoff = sub * RT8 + core * HT
    pltpu.sync_copy(sp_hbm.at[pl.ds(hoff, HT)], ssum.at[pl.ds(0, HT)])

    def accw(w, _):
        pltpu.sync_copy(sp_hbm.at[pl.ds(w * (N_PAD * 8) + hoff, HT)],
                        stmp.at[pl.ds(0, HT)])

        def vadd(j, _):
            jsl = pl.ds(j * 16, 16)
            ssum[jsl] = ssum[jsl] + stmp[jsl]
            return 0

        lax.fori_loop(0, HT // 16, vadd, 0, unroll=8)
        return 0

    lax.fori_loop(1, NW, accw, 0)
    pltpu.sync_copy(ssum.at[pl.ds(0, HT)], ssum_out.at[pl.ds(hoff, HT)])

    plsc.subcore_barrier()

    iota = lax.iota(jnp.int32, 16)
    base = wid * PER_W

    def chunk(i, _):
        off = base + i * K
        d1 = pltpu.async_copy(src_hbm.at[pl.ds(off, K)], srcv, gsem)
        d2 = pltpu.async_copy(dst_hbm.at[pl.ds(off, K)], dstv, gsem2)
        d3 = pltpu.async_copy(ee_hbm.at[pl.ds(off * 8, K * 8)], eev, gsem3)
        d1.wait()
        dh = pltpu.async_copy(h_hbm.at[srcv], hrows, gsem)
        d2.wait()
        d3.wait()
        dh.wait()

        def wrow(k, _):
            rowk = jnp.full((16,), k, jnp.int32)
            rowk8 = lax.shift_left(rowk, 3)
            for m in range(8):
                hd = (16 * m) // D
                colh = iota + (16 * m)
                a = plsc.load_gather(eev, [rowk8 + hd])
                hv = plsc.load_gather(hrows, [rowk, colh])
                plsc.store_scatter(hrows, [rowk, colh], hv * a)
            return 0

        lax.fori_loop(0, K, wrow, 0, unroll=2)
        pltpu.sync_copy(hrows, acc_sh.at[dstv], add=True)
        return 0

    lax.fori_loop(0, CHUNKS, chunk, 0)
    plsc.subcore_barrier()
    pltpu.sync_copy(acc_sh.at[sl], acc_out.at[core, sl])


def _sc_b(H, h, ee, src, dst, zf, s_parts):
    mesh = plsc.VectorSubcoreMesh(core_axis_name="c", subcore_axis_name="s")
    kfn = pl.kernel(
        functools.partial(_sc_b_body, H),
        out_type=(
            jax.ShapeDtypeStruct((NC, N_PAD, F), jnp.float32),
            jax.ShapeDtypeStruct((N_PAD * 8,), jnp.float32),
        ),
        mesh=mesh,
        compiler_params=_SC_PARAMS,
        scratch_types=[
            pltpu.VMEM_SHARED((N_PAD, F), jnp.float32),
            pltpu.VMEM((K,), jnp.int32),
            pltpu.VMEM((K,), jnp.int32),
            pltpu.VMEM((K * 8,), jnp.float32),
            pltpu.VMEM((K, F), jnp.float32),
            pltpu.VMEM((ROWS_PER_TILE * 8,), jnp.float32),
            pltpu.VMEM((ROWS_PER_TILE * 8,), jnp.float32),
            pltpu.SemaphoreType.DMA,
            pltpu.SemaphoreType.DMA,
            pltpu.SemaphoreType.DMA,
        ],
    )
    return kfn(h, ee, src, dst, zf, s_parts)


def _make_acat(a_s, a_d):
    Hh, Dd = a_s.shape
    rows = jnp.arange(F)
    onehot = (rows[:, None] // Dd == jnp.arange(Hh)[None, :]).astype(jnp.float32)
    As = onehot * a_s.reshape(F, 1)
    Ad = onehot * a_d.reshape(F, 1)
    cat = jnp.concatenate([As, Ad], axis=1)
    return jnp.pad(cat, ((0, 0), (0, F - cat.shape[1])))


def _make_c16(colmax, H):
    ms = colmax[0, :H]
    md = colmax[0, H:2 * H]
    c = ms + md
    c = jnp.maximum(c, 0.2 * c)
    return jnp.tile(c, 16 // H)


def _layer(H, h, alcat, cm, src, dst, zs, zf):
    c16 = _make_c16(cm, H)
    ee, s_parts = _sc_a(H, alcat, src, dst, c16, zs)
    o_p, s_sum = _sc_b(H, h, ee, src, dst, zf, s_parts)
    return s_sum.reshape(N_PAD, 8)[:N], o_p[:, :N]


def kernel(x, edge_index, edge_attr, W_enc, as_enc, ad_enc, b_enc,
           W_h0, as_h0, ad_h0, b_h0, W_h1, as_h1, ad_h1, b_h1,
           W_h2, as_h2, ad_h2, b_h2, W_dec, as_dec, ad_dec, b_dec):
    loop = jnp.arange(N, dtype=jnp.int32)
    pad = jnp.arange(E_PAD - E_REAL, dtype=jnp.int32)
    src = jnp.concatenate([edge_index[0].astype(jnp.int32), loop, pad])
    dst = jnp.concatenate([edge_index[1].astype(jnp.int32), loop, pad])

    zs = jnp.zeros((N_PAD * 8,), jnp.float32)
    zf = jnp.zeros((N_PAD, F), jnp.float32)
    bmask8 = jnp.repeat(jnp.eye(8, dtype=jnp.float32), 16, axis=1)
    bmask1 = jnp.concatenate(
        [jnp.ones((1, F), jnp.float32), jnp.zeros((7, F), jnp.float32)])

    h, alcat, cm = _tc_first(x, W_enc, _make_acat(as_enc, ad_enc))
    s_p, o_p = _layer(8, h, alcat, cm, src, dst, zs, zf)

    for (Wl, asl, adl), bprev, act in (
            ((W_h0, as_h0, ad_h0), b_enc, True),
            ((W_h1, as_h1, ad_h1), b_h0, False),
            ((W_h2, as_h2, ad_h2), b_h1, False),
    ):
        h, alcat, cm = _tc_mid(o_p[0], o_p[1], s_p, bprev.reshape(1, F),
                               bmask8, Wl, _make_acat(asl, adl), act)
        s_p, o_p = _layer(8, h, alcat, cm, src, dst, zs, zf)

    h, alcat, cm = _tc_mid(o_p[0], o_p[1], s_p, b_h2.reshape(1, F),
                           bmask8, W_dec, _make_acat(as_dec, ad_dec), False)
    s_p, o_p = _layer(1, h, alcat, cm, src, dst, zs, zf)

    return _tc_final(o_p[0], o_p[1], s_p, b_dec.reshape(1, F), bmask1)
